# Initial kernel scaffold; baseline (speedup 1.0000x reference)
import jax
import jax.numpy as jnp
from jax.experimental import pallas as pl


def kernel(x, pos, edge_index, edge_attr, params):
    raise NotImplementedError("write your pallas kernel here")


# speedup vs baseline: 2.0097x; 2.0097x over previous
import functools

import jax
import jax.numpy as jnp
from jax import lax
from jax.experimental import pallas as pl
from jax.experimental.pallas import tpu as pltpu
from jax.experimental.pallas import tpu_sc as plsc

_N = 10000
_NA = 10240
_E = 160000
_E_PAD = 163840
_NW = 32
_DK = 4
_RH = 32
_MID = 16
_OUT_CH = 16
_EB = 2048


def _gather_rows(table, idx2d, m_rows):
    n, c = table.shape
    perw = m_rows // _NW
    ch = 1024
    nch = perw // ch
    assert nch * ch == perw
    mesh = plsc.VectorSubcoreMesh(core_axis_name="c", subcore_axis_name="s")

    @functools.partial(
        pl.kernel, mesh=mesh,
        out_type=jax.ShapeDtypeStruct((m_rows, c), jnp.float32),
        compiler_params=pltpu.CompilerParams(use_tc_tiling_on_sc=False),
        scratch_types=[
            pltpu.VMEM((ch // 128, 128), jnp.int32),
            pltpu.VMEM((ch, c), jnp.float32),
            pltpu.SemaphoreType.DMA,
        ],
    )
    def k(table_hbm, idx_hbm, out_hbm, idx_v, rows_v, sem):
        wid = lax.axis_index("s") * 2 + lax.axis_index("c")
        base = wid * perw

        def body(j, carry):
            r0 = pl.multiple_of((base + j * ch) // 128, 8)
            pltpu.sync_copy(idx_hbm.at[pl.ds(r0, ch // 128)], idx_v)
            handles = [
                pltpu.async_copy(table_hbm.at[idx_v.at[jj]],
                                 rows_v.at[pl.ds(jj * 128, 128)], sem)
                for jj in range(ch // 128)
            ]
            for h in handles:
                h.wait()
            o0 = pl.multiple_of(base + j * ch, 8)
            pltpu.sync_copy(rows_v, out_hbm.at[pl.ds(o0, ch)])
            return carry

        lax.fori_loop(0, nch, body, 0)

    return k(table, idx2d)


def _scatter_add_rows(vals, dst2d, zeros_nw):
    e_pad, w = vals.shape
    n = zeros_nw.shape[0]
    perw = e_pad // _NW
    rpt = n // 16
    mesh = plsc.VectorSubcoreMesh(core_axis_name="c", subcore_axis_name="s")

    @functools.partial(
        pl.kernel, mesh=mesh,
        out_type=jax.ShapeDtypeStruct((2, n, w), jnp.float32),
        compiler_params=pltpu.CompilerParams(use_tc_tiling_on_sc=False),
        scratch_types=[
            pltpu.VMEM((perw, w), jnp.float32),
            pltpu.VMEM((perw // 128, 128), jnp.int32),
            pltpu.VMEM_SHARED((n, w), jnp.float32),
        ],
    )
    def k(vals_hbm, dst_hbm, zeros_hbm, out_hbm, val_v, dst_v, shared):
        cid = lax.axis_index("c")
        sid = lax.axis_index("s")
        wid = sid * 2 + cid
        base = wid * perw
        z0 = pl.multiple_of(sid * rpt, 8)
        pltpu.sync_copy(zeros_hbm.at[pl.ds(z0, rpt)],
                        shared.at[pl.ds(z0, rpt)])
        plsc.subcore_barrier()
        pltpu.sync_copy(vals_hbm.at[pl.ds(pl.multiple_of(base, 8), perw)],
                        val_v)
        pltpu.sync_copy(dst_hbm.at[pl.ds(pl.multiple_of(base // 128, 8),
                                         perw // 128)], dst_v)

        def body(j, carry):
            pltpu.sync_copy(val_v.at[pl.ds(j * 128, 128)],
                            shared.at[dst_v.at[j]], add=True)
            return carry

        lax.fori_loop(0, perw // 128, body, 0)
        plsc.subcore_barrier()
        pltpu.sync_copy(shared.at[pl.ds(z0, rpt)],
                        out_hbm.at[cid].at[pl.ds(z0, rpt)])

    return k(vals, dst2d, zeros_nw)


def _full(shape):
    return pl.BlockSpec(shape, lambda i: (0,) * len(shape))


def _rfeat_body(ps_ref, pd_ref, ea_ref, out_ref):
    d = pd_ref[...] - ps_ref[...]
    r2 = jnp.sum(d * d, axis=1, keepdims=True)
    r = jnp.sqrt(r2 + 1e-8)
    lane = lax.broadcasted_iota(jnp.int32, out_ref.shape, 1)
    out_ref[...] = ea_ref[...] + r * (lane == 0).astype(jnp.float32)


def _tc_rfeat(posg, ea8):
    nb = _E_PAD // _EB
    return pl.pallas_call(
        _rfeat_body,
        grid=(nb,),
        in_specs=[
            pl.BlockSpec((_EB, 8), lambda i: (i, 0)),
            pl.BlockSpec((_EB, 8), lambda i: (i + nb, 0)),
            pl.BlockSpec((_EB, 8), lambda i: (i, 0)),
        ],
        out_specs=pl.BlockSpec((_EB, 8), lambda i: (i, 0)),
        out_shape=jax.ShapeDtypeStruct((_E_PAD, 8), jnp.float32),
    )(posg, posg, ea8)


def _dbf(a, b):
    return jnp.dot(a.astype(jnp.bfloat16), b.astype(jnp.bfloat16),
                   preferred_element_type=jnp.float32)


def _bf(x):
    return x.astype(jnp.bfloat16).astype(jnp.float32)


_CONTRACT = _bf


def _radial(rf, w1_ref, b1_ref, w2_ref, b2_ref):
    h1 = jnp.maximum(_dbf(rf, w1_ref[...]) + b1_ref[...], 0.0)
    return jnp.maximum(_dbf(h1, w2_ref[...]) + b2_ref[...], 0.0)


def _edge_mlp_body(rf_ref, hs_ref, hd_ref,
                   w1k_ref, b1k_ref, w2k_ref, b2k_ref, w3k_ref, b3k_ref,
                   w1v_ref, b1v_ref, w2v_ref, b2v_ref, w3v_ref, b3v_ref,
                   wq_ref, out_ref, mm_ref):
    rf = rf_ref[...]
    hs = hs_ref[...]
    h2k = _radial(rf, w1k_ref, b1k_ref, w2k_ref, b2k_ref)
    h2v = _radial(rf, w1v_ref, b1v_ref, w2v_ref, b2v_ref)
    q = _dbf(hd_ref[...], wq_ref[...])
    hsb = _CONTRACT(hs)
    lane4 = lax.broadcasted_iota(jnp.int32, q.shape, 1)
    kcat = jnp.zeros_like(q)
    vs = []
    for o in range(_DK):
        ko = jnp.sum(_CONTRACT(_dbf(h2k, w3k_ref[o]) + b3k_ref[o]) * hsb,
                     axis=1, keepdims=True)
        kcat = kcat + ko * (lane4 == o).astype(jnp.float32)
        vo = jnp.sum(_CONTRACT(_dbf(h2v, w3v_ref[o]) + b3v_ref[o]) * hsb,
                     axis=1, keepdims=True)
        vs.append(vo)
    logits = jnp.sum(q * kcat, axis=1, keepdims=True) * 0.5
    eidx = pl.program_id(0) * _EB + lax.broadcasted_iota(
        jnp.int32, logits.shape, 0)
    mask = eidx < _E
    maskf = mask.astype(jnp.float32)
    lane8 = lax.broadcasted_iota(jnp.int32, out_ref.shape, 1)
    out = logits * (lane8 == 0).astype(jnp.float32)
    for o in range(_DK):
        out = out + vs[o] * (lane8 == o + 1).astype(jnp.float32)
    out = out + maskf * (lane8 == 5).astype(jnp.float32)
    out_ref[...] = out
    neg = jnp.full(logits.shape, -3e38, jnp.float32)
    bmax = jnp.max(jnp.where(mask, logits, neg))
    bmin = jnp.max(jnp.where(mask, -logits, neg))
    l2 = lax.broadcasted_iota(jnp.int32, mm_ref.shape, 1)
    blk = (bmax * (l2 == 0).astype(jnp.float32)
           + bmin * (l2 == 1).astype(jnp.float32)
           + jnp.where(l2 >= 2, -3e38, 0.0))

    @pl.when(pl.program_id(0) == 0)
    def _():
        mm_ref[...] = jnp.full(mm_ref.shape, -3e38, jnp.float32)

    mm_ref[...] = jnp.maximum(mm_ref[...], blk)


def _tc_edge_mlp(rfeat8, hg, p, c_in):
    nb = _E_PAD // _EB
    w3k = p["kr"]["W3"].reshape(_RH, _DK, c_in).transpose(1, 0, 2)
    b3k = p["kr"]["b3"].reshape(_DK, 1, c_in)
    w3v = p["vr"]["W3"].reshape(_RH, _DK, c_in).transpose(1, 0, 2)
    b3v = p["vr"]["b3"].reshape(_DK, 1, c_in)
    wts = [
        jnp.pad(p["kr"]["W1"], ((0, 3), (0, 0))), p["kr"]["b1"].reshape(1, _RH),
        p["kr"]["W2"], p["kr"]["b2"].reshape(1, _RH), w3k, b3k,
        jnp.pad(p["vr"]["W1"], ((0, 3), (0, 0))), p["vr"]["b1"].reshape(1, _RH),
        p["vr"]["W2"], p["vr"]["b2"].reshape(1, _RH), w3v, b3v,
        p["Wq"],
    ]
    return pl.pallas_call(
        _edge_mlp_body,
        grid=(nb,),
        in_specs=[
            pl.BlockSpec((_EB, 8), lambda i: (i, 0)),
            pl.BlockSpec((_EB, c_in), lambda i: (i, 0)),
            pl.BlockSpec((_EB, c_in), lambda i: (i + nb, 0)),
        ] + [_full(w.shape) for w in wts],
        out_specs=(pl.BlockSpec((_EB, 8), lambda i: (i, 0)),
                   pl.BlockSpec((1, 8), lambda i: (0, 0))),
        out_shape=(jax.ShapeDtypeStruct((_E_PAD, 8), jnp.float32),
                   jax.ShapeDtypeStruct((1, 8), jnp.float32)),
    )(rfeat8, hg, hg, *wts)


def _bucket_body(divpow, first, ev_ref, log_ref, mm_ref, out_ref):
    l = ev_ref[...][:, 0:1]
    m_hi = mm_ref[0, 0]
    m_lo = -mm_ref[0, 1]
    wid = (m_hi - m_lo + 1.0) / (16.0 * divpow)
    lo = m_lo if first else log_ref[...][:, 0:1]
    b = jnp.clip(jnp.floor((l - lo) / wid), 0.0, 15.0).astype(jnp.int32)
    eidx = pl.program_id(0) * _EB + lax.broadcasted_iota(jnp.int32, b.shape, 0)
    maskf = (eidx < _E).astype(jnp.float32)
    lane16 = lax.broadcasted_iota(jnp.int32, out_ref.shape, 1)
    out_ref[...] = (lane16 == b).astype(jnp.float32) * maskf


def _tc_bucket(ev1, log_, mm, divpow):
    nb = _E_PAD // _EB
    first = log_ is None
    if first:
        log_ = ev1
    return pl.pallas_call(
        functools.partial(_bucket_body, float(divpow), first),
        grid=(nb,),
        in_specs=[
            pl.BlockSpec((_EB, 8), lambda i: (i, 0)),
            pl.BlockSpec((_EB, 8), lambda i: (i, 0)),
            _full((1, 8)),
        ],
        out_specs=pl.BlockSpec((_EB, 16), lambda i: (i, 0)),
        out_shape=jax.ShapeDtypeStruct((_E_PAD, 16), jnp.float32),
    )(ev1, log_, mm)


def _lo_update_body(divpow, first, delta, acc_ref, mm_ref, lo_ref, out_ref):
    cnt = acc_ref[0] + acc_ref[1]
    m_hi = mm_ref[0, 0]
    m_lo = -mm_ref[0, 1]
    wid = (m_hi - m_lo + 1.0) / (16.0 * divpow)
    lane16 = lax.broadcasted_iota(jnp.int32, cnt.shape, 1)
    h1 = jnp.max(jnp.where(cnt > 0.0, (lane16 + 1).astype(jnp.float32), 0.0),
                 axis=1, keepdims=True)
    lo = jnp.zeros(out_ref.shape, jnp.float32) + m_lo if first else lo_ref[...]
    out_ref[...] = lo + jnp.maximum(h1 - delta, 0.0) * wid


def _tc_lo_update(acc2, mm, lo, divpow, final):
    first = lo is None
    if first:
        lo = mm
        lo_spec = _full((1, 8))
    else:
        lo_spec = _full((_N, 8))
    return pl.pallas_call(
        functools.partial(_lo_update_body, float(divpow), first,
                          0.0 if final else 1.0),
        grid=(1,),
        in_specs=[_full((2, _N, 16)), _full((1, 8)), lo_spec],
        out_specs=_full((_N, 8)),
        out_shape=jax.ShapeDtypeStruct((_N, 8), jnp.float32),
    )(acc2, mm, lo)


def _a_pass_body(ev_ref, log_ref, out_ref):
    ev = ev_ref[...]
    l = ev[:, 0:1]
    lo = log_ref[...][:, 0:1]
    eidx = pl.program_id(0) * _EB + lax.broadcasted_iota(jnp.int32, l.shape, 0)
    maskf = (eidx < _E).astype(jnp.float32)
    a = jnp.exp(l - lo) * maskf
    lane8 = lax.broadcasted_iota(jnp.int32, out_ref.shape, 1)
    out = a * (lane8 == 0).astype(jnp.float32)
    for o in range(_DK):
        vo = jnp.sum(ev * (lane8 == o + 1).astype(jnp.float32),
                     axis=1, keepdims=True)
        out = out + (a * vo) * (lane8 == o + 1).astype(jnp.float32)
    out = out + maskf * (lane8 == 5).astype(jnp.float32)
    out_ref[...] = out


def _tc_a_pass(ev1, log_):
    nb = _E_PAD // _EB
    return pl.pallas_call(
        _a_pass_body,
        grid=(nb,),
        in_specs=[
            pl.BlockSpec((_EB, 8), lambda i: (i, 0)),
            pl.BlockSpec((_EB, 8), lambda i: (i, 0)),
        ],
        out_specs=pl.BlockSpec((_EB, 8), lambda i: (i, 0)),
        out_shape=jax.ShapeDtypeStruct((_E_PAD, 8), jnp.float32),
    )(ev1, log_)


def _node_body(acc_ref, h_ref, wproj_ref, wskip_ref, g_ref, b_ref,
               h_out_ref, deg_ref):
    acc = acc_ref[0] + acc_ref[1]
    z = acc[:, 0:1]
    lane8 = lax.broadcasted_iota(jnp.int32, acc.shape, 1)
    deg_ref[...] = jnp.sum(acc * (lane8 == 5).astype(jnp.float32),
                           axis=1, keepdims=True)
    accn = acc / (z + 1e-9)
    hnew = _dbf(accn, wproj_ref[...]) + _dbf(h_ref[...], wskip_ref[...])
    h_out_ref[...] = (jnp.maximum(jnp.abs(hnew) * g_ref[...] + b_ref[...], 0.0)
                      * jnp.sign(hnew))


def _tc_node(acc2, h, p):
    c_in = h.shape[1]
    wproj8 = jnp.zeros((8, _MID), jnp.float32).at[1:5].set(p["Wproj"])
    wts = [wproj8, p["Wskip"], p["norm_g"].reshape(1, _MID),
           p["norm_b"].reshape(1, _MID)]
    return pl.pallas_call(
        _node_body,
        grid=(1,),
        in_specs=[_full((2, _N, 8)), _full((_N, c_in))] +

                 [_full(w.shape) for w in wts],
        out_specs=(_full((_N, _MID)), _full((_N, 1))),
        out_shape=(jax.ShapeDtypeStruct((_N, _MID), jnp.float32),
                   jax.ShapeDtypeStruct((_N, 1), jnp.float32)),
    )(acc2, h, *wts)


def _edge_conv_body(rf_ref, hs_ref, w1_ref, b1_ref, w2_ref, b2_ref,
                    w3_ref, b3_ref, out_ref):
    rf = rf_ref[...]
    hs = hs_ref[...]
    h2 = _radial(rf, w1_ref, b1_ref, w2_ref, b2_ref)
    lane = lax.broadcasted_iota(jnp.int32, out_ref.shape, 1)
    eidx = pl.program_id(0) * _EB + lax.broadcasted_iota(
        jnp.int32, (out_ref.shape[0], 1), 0)
    maskf = (eidx < _E).astype(jnp.float32)
    hsb = _CONTRACT(hs)
    out = jnp.zeros(out_ref.shape, jnp.float32)
    for o in range(_OUT_CH):
        mo = jnp.sum(_CONTRACT(_dbf(h2, w3_ref[o]) + b3_ref[o]) * hsb,
                     axis=1, keepdims=True)
        out = out + mo * (lane == o).astype(jnp.float32)
    out_ref[...] = out * maskf


def _tc_edge_conv(rfeat8, hg, pc):
    nb = _E_PAD // _EB
    w3 = pc["r"]["W3"].reshape(_RH, _OUT_CH, _MID).transpose(1, 0, 2)
    b3 = pc["r"]["b3"].reshape(_OUT_CH, 1, _MID)
    wts = [jnp.pad(pc["r"]["W1"], ((0, 3), (0, 0))),
           pc["r"]["b1"].reshape(1, _RH),
           pc["r"]["W2"], pc["r"]["b2"].reshape(1, _RH), w3, b3]
    return pl.pallas_call(
        _edge_conv_body,
        grid=(nb,),
        in_specs=[
            pl.BlockSpec((_EB, 8), lambda i: (i, 0)),
            pl.BlockSpec((_EB, _MID), lambda i: (i, 0)),
        ] + [_full(w.shape) for w in wts],
        out_specs=pl.BlockSpec((_EB, _OUT_CH), lambda i: (i, 0)),
        out_shape=jax.ShapeDtypeStruct((_E_PAD, _OUT_CH), jnp.float32),
    )(rfeat8, hg, *wts)


def _final_body(acc_ref, deg_ref, h_ref, wself_ref, w1_ref, b1_ref,
                w2_ref, b2_ref, out_ref):
    acc = acc_ref[0] + acc_ref[1]
    hf = (acc / jnp.maximum(deg_ref[...], 1.0)
          + _dbf(h_ref[...], wself_ref[...]))
    gm = jnp.mean(hf, axis=0, keepdims=True)
    gm = jnp.maximum(_dbf(gm, w1_ref[...]) + b1_ref[...], 0.0)
    out_ref[...] = _dbf(gm, w2_ref[...]) + b2_ref[...]


def _tc_final(acc2, deg, h, pc, fc):
    wts = [pc["Wself"], fc["W1"], fc["b1"].reshape(1, _OUT_CH),
           fc["W2"], fc["b2"].reshape(1, 1)]
    return pl.pallas_call(
        _final_body,
        grid=(1,),
        in_specs=[_full((2, _N, _OUT_CH)), _full((_N, 1)),
                  _full((_N, _MID))] + [_full(w.shape) for w in wts],
        out_specs=_full((1, 1)),
        out_shape=jax.ShapeDtypeStruct((1, 1), jnp.float32),
    )(acc2, deg, h, *wts)


def kernel(x, pos, edge_index, edge_attr, params):
    src = edge_index[0]
    dst = edge_index[1]
    pad = _E_PAD - _E
    src_p = jnp.concatenate([src, jnp.zeros((pad,), jnp.int32)])
    dst_p = jnp.concatenate([dst, jnp.zeros((pad,), jnp.int32)])
    idx_cat2d = jnp.concatenate([src_p, dst_p]).reshape(-1, 128)
    src2d = src_p.reshape(-1, 128)
    dst2d = dst_p.reshape(-1, 128)
    pos_pad = jnp.pad(pos, ((0, 0), (0, 5)))
    ea8 = jnp.pad(edge_attr, ((0, pad), (1, 3)))
    zeros8 = jnp.zeros((_NA, 8), jnp.float32)
    zeros16 = jnp.zeros((_NA, _OUT_CH), jnp.float32)

    posg = _gather_rows(pos_pad, idx_cat2d, 2 * _E_PAD)
    rfeat8 = _tc_rfeat(posg, ea8)

    h = x
    deg = None
    for li, p in enumerate(params["layers"]):
        c_in = h.shape[1]
        hg = _gather_rows(h, idx_cat2d, 2 * _E_PAD)
        ev1, mm = _tc_edge_mlp(rfeat8, hg, p, c_in)
        rounds = 2 if li == 0 else 4
        lo = None
        for r in range(1, rounds + 1):
            log_ = _gather_rows(lo, dst2d, _E_PAD) if r > 1 else None
            oh = _tc_bucket(ev1, log_, mm, 16 ** (r - 1))
            accb = _scatter_add_rows(oh, dst2d, zeros16)
            lo = _tc_lo_update(accb, mm, lo, 16 ** (r - 1), r == rounds)
        log_ = _gather_rows(lo, dst2d, _E_PAD)
        ev = _tc_a_pass(ev1, log_)
        acc2 = _scatter_add_rows(ev, dst2d, zeros8)
        h, deg_l = _tc_node(acc2, h, p)
        if li == 0:
            deg = deg_l

    hg = _gather_rows(h, src2d, _E_PAD)
    msg = _tc_edge_conv(rfeat8, hg, params["conv"])
    acc2 = _scatter_add_rows(msg, dst2d, zeros16)
    return _tc_final(acc2, deg, h, params["conv"], params["fc"])
